# baseline (speedup 1.0000x reference)
import functools

import jax
import jax.numpy as jnp
from jax import lax
from jax.experimental import pallas as pl
from jax.experimental.pallas import tpu as pltpu
from jax.experimental.pallas import tpu_sc as plsc

N = 10000
E = 320000
L = 4
DIN = 128
DOUT = 128

NC = 2
NS = 16
LPC = L // NC

NP = 10240
ZROW = NP - 2
NPA = 10112
DUMP = NPA - 1
GW = 48
G = 420
CH = 6
NCH = G // CH
RB = 6
EPL = G * GW * NS
GWD = 120
GD = EPL // (GWD * NS)
STRIPE = NP // NS
STRIPA = NPA // NS

@functools.cache
def _sc_kernels():
    mesh = plsc.VectorSubcoreMesh(core_axis_name="c", subcore_axis_name="s",
                                  num_cores=NC, num_subcores=NS)
    deg_k = functools.partial(
        pl.kernel,
        out_type=jax.ShapeDtypeStruct((L, NP), jnp.float32),
        mesh=mesh,
        scratch_types=[
            pltpu.VMEM_SHARED((NP,), jnp.float32),
            pltpu.VMEM_SHARED((NP,), jnp.float32),
            pltpu.VMEM((GD, GWD), jnp.int32),
            pltpu.VMEM((GWD,), jnp.float32),
        ],
    )(_deg_body)
    agg_k = functools.partial(
        pl.kernel,
        out_type=jax.ShapeDtypeStruct((L, NPA, DOUT), jnp.float32),
        mesh=mesh,
        scratch_types=[
            pltpu.VMEM_SHARED((NPA, DOUT), jnp.float32),
            pltpu.VMEM((CH, GW), jnp.int32),
            pltpu.VMEM((CH, GW), jnp.int32),
            pltpu.VMEM((CH, GW), jnp.int32),
            [pltpu.VMEM((GW, DOUT), jnp.float32) for _ in range(RB)],
            [pltpu.SemaphoreType.DMA for _ in range(RB)],
        ],
    )(_agg_body)
    return deg_k, agg_k


def _deg_body(dst_hbm, z1_hbm, ones_hbm, deg_hbm, acc0, acc1, dstbuf, ones_v):
    c = lax.axis_index("c")
    s = lax.axis_index("s")
    base = s * STRIPE
    pltpu.sync_copy(z1_hbm.at[pl.ds(base, STRIPE)], acc0.at[pl.ds(base, STRIPE)])
    pltpu.sync_copy(z1_hbm.at[pl.ds(base, STRIPE)], acc1.at[pl.ds(base, STRIPE)])
    pltpu.sync_copy(ones_hbm, ones_v)
    plsc.subcore_barrier()
    for i, acc in enumerate((acc0, acc1)):
        layer = c * LPC + i
        pltpu.sync_copy(dst_hbm.at[layer, s], dstbuf)

        def body(j, carry, acc=acc):
            pltpu.sync_copy(ones_v, acc.at[dstbuf.at[j]], add=True)
            return carry

        lax.fori_loop(0, GD, body, 0)
    plsc.subcore_barrier()
    for i, acc in enumerate((acc0, acc1)):
        layer = c * LPC + i
        pltpu.sync_copy(acc.at[pl.ds(base, STRIPE)],
                        deg_hbm.at[layer, pl.ds(base, STRIPE)])


def _agg_body(srcoff_hbm, dst_hbm, g2_hbm, z2_hbm, agg_hbm,
              acc, srcbuf, dstA, dstB, rows, sems):
    c = lax.axis_index("c")
    s = lax.axis_index("s")
    base = s * STRIPA

    def wait_buf(b):
        pltpu.make_async_copy(g2_hbm.at[srcbuf.at[0]], rows[b], sems[b]).wait()

    def run_chunk(srcslc, dstslc, dstbuf, first):
        pltpu.sync_copy(srcslc, srcbuf)
        pltpu.sync_copy(dstslc, dstbuf)
        for j in range(CH):
            b = j % RB
            if not (first and j < RB):
                wait_buf(b)
            pltpu.async_copy(g2_hbm.at[srcbuf.at[j]], rows[b], sems[b])
            if j % RB == RB - 1:
                for jj in range(j - (RB - 1), j + 1):
                    bb = jj % RB
                    wait_buf(bb)
                    pltpu.async_copy(rows[bb], acc.at[dstbuf.at[jj]],
                                     sems[bb], add=True)

    for i in range(LPC):
        layer = c * LPC + i
        row0 = s * NCH
        pltpu.sync_copy(z2_hbm.at[pl.ds(base, STRIPA)],
                        acc.at[pl.ds(base, STRIPA)])
        plsc.subcore_barrier()

        run_chunk(srcoff_hbm.at[layer, row0], dst_hbm.at[layer, row0],
                  dstA, first=True)

        def pair(k2, carry):
            k = 1 + 2 * k2
            run_chunk(srcoff_hbm.at[layer, row0 + k],
                      dst_hbm.at[layer, row0 + k], dstB, first=False)
            run_chunk(srcoff_hbm.at[layer, row0 + k + 1],
                      dst_hbm.at[layer, row0 + k + 1], dstA, first=False)
            return carry

        lax.fori_loop(0, (NCH - 2) // 2, pair, 0)
        run_chunk(srcoff_hbm.at[layer, row0 + NCH - 1],
                  dst_hbm.at[layer, row0 + NCH - 1], dstB, first=False)
        for b in range(RB):
            wait_buf(b)
        plsc.subcore_barrier()
        pltpu.sync_copy(acc.at[pl.ds(base, STRIPA)],
                        agg_hbm.at[layer, pl.ds(base, STRIPA)])


def _g_body(x_ref, w_ref, deg_ref, g_ref):
    dis = lax.rsqrt(deg_ref[0] + 1.0)
    h = jnp.dot(x_ref[...], w_ref[0], preferred_element_type=jnp.float32)
    g_ref[0] = h * dis


_g_matmul = pl.pallas_call(
    _g_body,
    grid=(L,),
    in_specs=[
        pl.BlockSpec((NP, DIN), lambda l: (0, 0)),
        pl.BlockSpec((1, DIN, DOUT), lambda l: (l, 0, 0)),
        pl.BlockSpec((1, NP, 1), lambda l: (l, 0, 0)),
    ],
    out_specs=pl.BlockSpec((1, NP, DOUT), lambda l: (l, 0, 0)),
    out_shape=jax.ShapeDtypeStruct((L, NP, DOUT), jnp.float32),
)


def _f_body(agg_ref, deg_ref, x_ref, w_ref, b_ref, out_ref):
    lidx = pl.program_id(0)
    dis = lax.rsqrt(deg_ref[0] + 1.0)
    h = jnp.dot(x_ref[...], w_ref[0], preferred_element_type=jnp.float32)
    t = agg_ref[0] * dis + h * (dis * dis) + b_ref[0]

    @pl.when(lidx == 0)
    def _init():
        out_ref[...] = t

    @pl.when(lidx != 0)
    def _accum():
        out_ref[...] = out_ref[...] + t


_final = pl.pallas_call(
    _f_body,
    grid=(L,),
    in_specs=[
        pl.BlockSpec((1, NPA, DOUT), lambda l: (l, 0, 0)),
        pl.BlockSpec((1, NPA, 1), lambda l: (l, 0, 0)),
        pl.BlockSpec((NPA, DIN), lambda l: (0, 0)),
        pl.BlockSpec((1, DIN, DOUT), lambda l: (l, 0, 0)),
        pl.BlockSpec((1, 1, DOUT), lambda l: (l, 0, 0)),
    ],
    out_specs=pl.BlockSpec((NPA, DOUT), lambda l: (0, 0)),
    out_shape=jax.ShapeDtypeStruct((NPA, DOUT), jnp.float32),
)


def kernel(x, edge_index, W, b):
    i32 = jnp.int32
    f32 = jnp.float32
    src = edge_index[:, 0, :]
    dst = edge_index[:, 1, :]
    pad = EPL - E
    loff = (jnp.arange(L, dtype=i32) * NP)[:, None]
    srcp = (jnp.concatenate([src, jnp.full((L, pad), ZROW, i32)], axis=1)
            + loff)
    dstp = jnp.concatenate([dst, jnp.full((L, pad), DUMP, i32)], axis=1)
    xp = jnp.pad(x, ((0, NP - N), (0, 0)))
    z1 = jnp.zeros((NP,), f32)
    z2 = jnp.zeros((NP, DOUT), f32)
    ones = jnp.ones((GWD,), f32)

    deg_k, agg_k = _sc_kernels()
    deg = deg_k(dstp.reshape(L, NS, GD, GWD), z1, ones)
    deg3 = deg.reshape(L, NP, 1)
    g = _g_matmul(xp, W, deg3)
    agg = agg_k(srcp.reshape(L, NS * NCH, CH, GW),
                dstp.reshape(L, NS * NCH, CH, GW),
                g.reshape(L * NP, DOUT), z2)
    outp = _final(agg, deg3, xp, W, b.reshape(L, 1, DOUT))
    return outp[:N]


# speedup vs baseline: 1.0977x; 1.0977x over previous
import functools

import jax
import jax.numpy as jnp
from jax import lax
from jax.experimental import pallas as pl
from jax.experimental.pallas import tpu as pltpu
from jax.experimental.pallas import tpu_sc as plsc

N = 10000
E = 320000
L = 4
DIN = 128
DOUT = 128

NC = 2
NS = 16
LPC = L // NC

NP = 10240
ZROW = NP - 2
NPA = 10112
DUMP = NPA - 1
GW = 120
G = 168
CH = 6
NCH = G // CH
RB = 3
EPL = G * GW * NS
GWD = 120
GD = EPL // (GWD * NS)
STRIPE = NP // NS
STRIPA = NPA // NS

@functools.cache
def _sc_kernels():
    mesh = plsc.VectorSubcoreMesh(core_axis_name="c", subcore_axis_name="s",
                                  num_cores=NC, num_subcores=NS)
    deg_k = functools.partial(
        pl.kernel,
        out_type=jax.ShapeDtypeStruct((L, NP), jnp.float32),
        mesh=mesh,
        scratch_types=[
            pltpu.VMEM_SHARED((NP,), jnp.float32),
            pltpu.VMEM_SHARED((NP,), jnp.float32),
            pltpu.VMEM((GD, GWD), jnp.int32),
            pltpu.VMEM((GWD,), jnp.float32),
            pltpu.SemaphoreType.DMA,
        ],
    )(_deg_body)
    agg_k = functools.partial(
        pl.kernel,
        out_type=jax.ShapeDtypeStruct((L, NPA, DOUT), jnp.float32),
        mesh=mesh,
        scratch_types=[
            pltpu.VMEM_SHARED((NPA, DOUT), jnp.float32),
            pltpu.VMEM((CH, GW), jnp.int32),
            pltpu.VMEM((CH, GW), jnp.int32),
            pltpu.VMEM((CH, GW), jnp.int32),
            [pltpu.VMEM((GW, DOUT), jnp.float32) for _ in range(RB)],
            [pltpu.SemaphoreType.DMA for _ in range(RB)],
        ],
    )(_agg_body)
    return deg_k, agg_k


def _deg_body(dst_hbm, z1_hbm, ones_hbm, deg_hbm, acc0, acc1, dstbuf, ones_v,
              dsem):
    c = lax.axis_index("c")
    s = lax.axis_index("s")
    base = s * STRIPE
    pltpu.sync_copy(z1_hbm.at[pl.ds(base, STRIPE)], acc0.at[pl.ds(base, STRIPE)])
    pltpu.sync_copy(z1_hbm.at[pl.ds(base, STRIPE)], acc1.at[pl.ds(base, STRIPE)])
    pltpu.sync_copy(ones_hbm, ones_v)
    plsc.subcore_barrier()
    for i, acc in enumerate((acc0, acc1)):
        layer = c * LPC + i
        pltpu.sync_copy(dst_hbm.at[layer, s], dstbuf)

        def body(k, carry, acc=acc):
            for jj in range(8):
                pltpu.async_copy(ones_v, acc.at[dstbuf.at[k * 8 + jj]],
                                 dsem, add=True)
            for jj in range(8):
                pltpu.make_async_copy(ones_v, acc.at[dstbuf.at[0]],
                                      dsem).wait()
            return carry

        lax.fori_loop(0, GD // 8, body, 0)
    plsc.subcore_barrier()
    for i, acc in enumerate((acc0, acc1)):
        layer = c * LPC + i
        pltpu.sync_copy(acc.at[pl.ds(base, STRIPE)],
                        deg_hbm.at[layer, pl.ds(base, STRIPE)])


def _agg_body(srcoff_hbm, dst_hbm, g2_hbm, z2_hbm, agg_hbm,
              acc, srcbuf, dstA, dstB, rows, sems):
    c = lax.axis_index("c")
    s = lax.axis_index("s")
    base = s * STRIPA

    def wait_buf(b):
        pltpu.make_async_copy(g2_hbm.at[srcbuf.at[0]], rows[b], sems[b]).wait()

    def run_chunk(srcslc, dstslc, dstbuf, first):
        pltpu.sync_copy(srcslc, srcbuf)
        pltpu.sync_copy(dstslc, dstbuf)
        for j in range(CH):
            b = j % RB
            if not (first and j < RB):
                wait_buf(b)
            pltpu.async_copy(g2_hbm.at[srcbuf.at[j]], rows[b], sems[b])
            if j % RB == RB - 1:
                for jj in range(j - (RB - 1), j + 1):
                    bb = jj % RB
                    wait_buf(bb)
                    pltpu.async_copy(rows[bb], acc.at[dstbuf.at[jj]],
                                     sems[bb], add=True)

    for i in range(LPC):
        layer = c * LPC + i
        row0 = s * NCH
        pltpu.sync_copy(z2_hbm.at[pl.ds(base, STRIPA)],
                        acc.at[pl.ds(base, STRIPA)])
        plsc.subcore_barrier()

        run_chunk(srcoff_hbm.at[layer, row0], dst_hbm.at[layer, row0],
                  dstA, first=True)

        def pair(k2, carry):
            k = 1 + 2 * k2
            run_chunk(srcoff_hbm.at[layer, row0 + k],
                      dst_hbm.at[layer, row0 + k], dstB, first=False)
            run_chunk(srcoff_hbm.at[layer, row0 + k + 1],
                      dst_hbm.at[layer, row0 + k + 1], dstA, first=False)
            return carry

        lax.fori_loop(0, (NCH - 2) // 2, pair, 0)
        run_chunk(srcoff_hbm.at[layer, row0 + NCH - 1],
                  dst_hbm.at[layer, row0 + NCH - 1], dstB, first=False)
        for b in range(RB):
            wait_buf(b)
        plsc.subcore_barrier()
        pltpu.sync_copy(acc.at[pl.ds(base, STRIPA)],
                        agg_hbm.at[layer, pl.ds(base, STRIPA)])


def _g_body(x_ref, w_ref, deg_ref, g_ref):
    dis = lax.rsqrt(deg_ref[0] + 1.0)
    h = jnp.dot(x_ref[...], w_ref[0], preferred_element_type=jnp.float32)
    g_ref[0] = h * dis


_g_matmul = pl.pallas_call(
    _g_body,
    grid=(L,),
    in_specs=[
        pl.BlockSpec((NP, DIN), lambda l: (0, 0)),
        pl.BlockSpec((1, DIN, DOUT), lambda l: (l, 0, 0)),
        pl.BlockSpec((1, NP, 1), lambda l: (l, 0, 0)),
    ],
    out_specs=pl.BlockSpec((1, NP, DOUT), lambda l: (l, 0, 0)),
    out_shape=jax.ShapeDtypeStruct((L, NP, DOUT), jnp.float32),
)


def _f_body(agg_ref, deg_ref, x_ref, w_ref, b_ref, out_ref):
    lidx = pl.program_id(0)
    dis = lax.rsqrt(deg_ref[0] + 1.0)
    h = jnp.dot(x_ref[...], w_ref[0], preferred_element_type=jnp.float32)
    t = agg_ref[0] * dis + h * (dis * dis) + b_ref[0]

    @pl.when(lidx == 0)
    def _init():
        out_ref[...] = t

    @pl.when(lidx != 0)
    def _accum():
        out_ref[...] = out_ref[...] + t


_final = pl.pallas_call(
    _f_body,
    grid=(L,),
    in_specs=[
        pl.BlockSpec((1, NPA, DOUT), lambda l: (l, 0, 0)),
        pl.BlockSpec((1, NPA, 1), lambda l: (l, 0, 0)),
        pl.BlockSpec((NPA, DIN), lambda l: (0, 0)),
        pl.BlockSpec((1, DIN, DOUT), lambda l: (l, 0, 0)),
        pl.BlockSpec((1, 1, DOUT), lambda l: (l, 0, 0)),
    ],
    out_specs=pl.BlockSpec((NPA, DOUT), lambda l: (0, 0)),
    out_shape=jax.ShapeDtypeStruct((NPA, DOUT), jnp.float32),
)


def kernel(x, edge_index, W, b):
    i32 = jnp.int32
    f32 = jnp.float32
    src = edge_index[:, 0, :]
    dst = edge_index[:, 1, :]
    pad = EPL - E
    loff = (jnp.arange(L, dtype=i32) * NP)[:, None]
    srcp = (jnp.concatenate([src, jnp.full((L, pad), ZROW, i32)], axis=1)
            + loff)
    dstp = jnp.concatenate([dst, jnp.full((L, pad), DUMP, i32)], axis=1)
    xp = jnp.pad(x, ((0, NP - N), (0, 0)))
    z1 = jnp.zeros((NP,), f32)
    z2 = jnp.zeros((NP, DOUT), f32)
    ones = jnp.ones((GWD,), f32)

    deg_k, agg_k = _sc_kernels()
    deg = deg_k(dstp.reshape(L, NS, GD, GWD), z1, ones)
    deg3 = deg.reshape(L, NP, 1)
    g = _g_matmul(xp, W, deg3)
    agg = agg_k(srcp.reshape(L, NS * NCH, CH, GW),
                dstp.reshape(L, NS * NCH, CH, GW),
                g.reshape(L * NP, DOUT), z2)
    outp = _final(agg, deg3, xp, W, b.reshape(L, 1, DOUT))
    return outp[:N]
